# Initial kernel scaffold; baseline (speedup 1.0000x reference)
import jax
import jax.numpy as jnp
from jax.experimental import pallas as pl


def kernel(X, table):
    raise NotImplementedError("write your pallas kernel here")


# speedup vs baseline: 27.5901x; 27.5901x over previous
import jax
import jax.numpy as jnp
from jax import lax
from jax.experimental import pallas as pl
from jax.experimental.pallas import tpu as pltpu
from jax.experimental.pallas import tpu_sc as plsc

_RES1 = 511.0
_P1 = 2654435761
_P2 = 805459861
_TMASK = 2**22 - 1

_NC, _NS = 2, 16
_NW = _NC * _NS
_C = 2048


def _tec_body(xt_hbm, table_hbm, out_hbm, xbuf, idxbuf, rows, sem):
    n = out_hbm.shape[0]
    n_w = n // _NW
    n_chunks = n_w // _C
    wid = lax.axis_index("s") * _NC + lax.axis_index("c")

    def chunk_body(i, carry):
        base = wid * n_w + i * _C
        pltpu.sync_copy(xt_hbm.at[pl.ds(base, _C)], xbuf.at[pl.ds(0, _C)])
        pltpu.sync_copy(xt_hbm.at[pl.ds(n + base, _C)], xbuf.at[pl.ds(_C, _C)])
        pltpu.sync_copy(xt_hbm.at[pl.ds(2 * n + base, _C)],
                        xbuf.at[pl.ds(2 * _C, _C)])

        def hash_body(j, carry):
            o = j * 16
            x = xbuf[pl.ds(o, 16)]
            y = xbuf[pl.ds(_C + o, 16)]
            z = xbuf[pl.ds(2 * _C + o, 16)]

            def p2i(v):
                v = jnp.minimum(jnp.maximum(v, -1.0), 1.0)
                v = (v + 1.0) / 2.0
                v = v * _RES1
                return v.astype(jnp.int32).astype(jnp.uint32)

            h = (p2i(x) ^ (p2i(y) * jnp.uint32(_P1))
                 ^ (p2i(z) * jnp.uint32(_P2)))
            h = h & jnp.uint32(_TMASK)
            idxbuf[pl.ds(o, 16)] = h.astype(jnp.int32)
            return carry

        lax.fori_loop(0, _C // 16, hash_body, 0)
        pltpu.async_copy(table_hbm.at[idxbuf], rows, sem).wait()
        pltpu.sync_copy(rows.at[:, pl.ds(0, 2)], out_hbm.at[pl.ds(base, _C)])
        return carry

    lax.fori_loop(0, n_chunks, chunk_body, 0)


def kernel(X, table):
    n = X.shape[0]
    f = table.shape[1]
    mesh = plsc.VectorSubcoreMesh(core_axis_name="c", subcore_axis_name="s")
    k = pl.kernel(
        _tec_body,
        out_type=jax.ShapeDtypeStruct((n, f), jnp.float32),
        mesh=mesh,
        scratch_types=[
            pltpu.VMEM((3 * _C,), jnp.float32),
            pltpu.VMEM((_C,), jnp.int32),
            pltpu.VMEM((_C, 16), jnp.float32),
            pltpu.SemaphoreType.DMA,
        ],
        compiler_params=pltpu.CompilerParams(use_tc_tiling_on_sc=False),
    )
    table16 = jnp.pad(table, ((0, 0), (0, 16 - f)))
    return k(X.T.reshape(-1), table16)
